# baseline (speedup 1.0000x reference)
import functools

import jax
import jax.numpy as jnp
from jax import lax
from jax.experimental import pallas as pl
from jax.experimental.pallas import tpu as pltpu
from jax.experimental.pallas import tpu_sc as plsc

_N = 10000
_E = 320000
_H = 128
_G = 16

_NC = 2
_NS = 16
_NW = _NC * _NS
_CH = 128
_EPW = _E // _NW
_CPW = 80
_WCH = 40
_EPWP = _CPW * _CH
_NPAD = 10112
_RPT = _NPAD // _NS
_DW = 16

@functools.cache
def _mesh():
    return plsc.VectorSubcoreMesh(core_axis_name="c", subcore_axis_name="s",
                                  num_cores=_NC, num_subcores=_NS)


def _sc_deg_body(dstp_hbm, out_hbm, dst_v, ones_v, zb_v, acc_sh):
    c = lax.axis_index("c")
    s = lax.axis_index("s")
    wid = s * _NC + c

    lane = lax.iota(jnp.int32, 16)
    onehot = jnp.where(lane == 0, 1.0, 0.0).astype(jnp.float32)
    z16 = jnp.zeros((16,), jnp.float32)

    def fill(r, carry):
        ones_v[r, :] = onehot
        zb_v[r, :] = z16
        return carry

    lax.fori_loop(0, _CH, fill, 0)

    base = s * _RPT
    nfull = _RPT // _CH
    for k in range(nfull):
        pltpu.sync_copy(zb_v, acc_sh.at[pl.ds(base + k * _CH, _CH)])
    rem = _RPT - nfull * _CH
    if rem:
        pltpu.sync_copy(zb_v.at[pl.ds(0, rem)],
                        acc_sh.at[pl.ds(base + nfull * _CH, rem)])

    pltpu.sync_copy(dstp_hbm.at[wid], dst_v)
    plsc.subcore_barrier()

    def step(j, carry):
        pltpu.sync_copy(ones_v, acc_sh.at[dst_v.at[j]], add=True)
        return carry

    lax.fori_loop(0, _CPW, step, 0)
    plsc.subcore_barrier()
    pltpu.sync_copy(acc_sh.at[pl.ds(base, _RPT)],
                    out_hbm.at[c, pl.ds(base, _RPT)])


@functools.cache
def _sc_deg():
    return pl.kernel(
        _sc_deg_body,
        out_type=jax.ShapeDtypeStruct((_NC, _NPAD, _DW), jnp.float32),
        mesh=_mesh(),
        scratch_types=[
            pltpu.VMEM((_CPW, _CH), jnp.int32),
            pltpu.VMEM((_CH, _DW), jnp.float32),
            pltpu.VMEM((_CH, _DW), jnp.float32),
            pltpu.VMEM_SHARED((_NPAD, _DW), jnp.float32),
        ],
    )


def _sc_msg_body(u_hbm, srcp_hbm, dstp_hbm, out_hbm,
                 src_v, dst_v, rows0_v, acc_sh, sem_g0):
    c = lax.axis_index("c")
    s = lax.axis_index("s")
    wid = s * _NC + c

    z16 = jnp.zeros((16,), jnp.float32)

    def zfill(r, carry):
        for cc in range(_H // 16):
            rows0_v[r, pl.ds(cc * 16, 16)] = z16
        return carry

    lax.fori_loop(0, _CH, zfill, 0)

    base = s * _RPT
    nfull = _RPT // _CH
    for k in range(nfull):
        pltpu.sync_copy(rows0_v, acc_sh.at[pl.ds(base + k * _CH, _CH)])
    rem = _RPT - nfull * _CH
    if rem:
        pltpu.sync_copy(rows0_v.at[pl.ds(0, rem)],
                        acc_sh.at[pl.ds(base + nfull * _CH, rem)])

    plsc.subcore_barrier()

    pltpu.sync_copy(srcp_hbm.at[wid], src_v)
    pltpu.sync_copy(dstp_hbm.at[wid], dst_v)

    def step(j, carry):
        pltpu.async_copy(u_hbm.at[src_v.at[j]], rows0_v, sem_g0).wait()
        pltpu.sync_copy(rows0_v, acc_sh.at[dst_v.at[j]], add=True)
        return carry

    lax.fori_loop(0, _CPW, step, 0)

    plsc.subcore_barrier()
    pltpu.sync_copy(acc_sh.at[pl.ds(base, _RPT)],
                    out_hbm.at[c, pl.ds(base, _RPT)])


@functools.cache
def _sc_msg():
    return pl.kernel(
        _sc_msg_body,
        out_type=jax.ShapeDtypeStruct((_NC, _NPAD, _H), jnp.float32),
        mesh=_mesh(),
        scratch_types=[
            pltpu.VMEM((_CPW, _CH), jnp.int32),
            pltpu.VMEM((_CPW, _CH), jnp.int32),
            pltpu.VMEM((_CH, _H), jnp.float32),
            pltpu.VMEM_SHARED((_NPAD, _H), jnp.float32),
            pltpu.SemaphoreType.DMA,
        ],
    )


def _tc_pre_body(lc, w1, b1, w2, b2, wg0, degp, u0_out, dinv_out):
    deg = (jnp.sum(degp[0, :_N, :], axis=-1, keepdims=True)
           + jnp.sum(degp[1, :_N, :], axis=-1, keepdims=True) + 1.0)
    dinv = lax.rsqrt(deg)
    h = jnp.maximum(lc[...] * w1[...] + b1[...], 0.0)
    h = jnp.dot(h, w2[...], preferred_element_type=jnp.float32) + b2[...]
    u0_out[...] = dinv * jnp.dot(h, wg0[...], preferred_element_type=jnp.float32)
    dinv_out[...] = dinv


def _ln_relu(agg, lng, lnb):
    m = jnp.mean(agg, axis=-1, keepdims=True)
    v = jnp.mean((agg - m) ** 2, axis=-1, keepdims=True)
    h = (agg - m) * lax.rsqrt(v + 1e-5) * lng + lnb
    return jnp.maximum(h, 0.0)


def _tc_mid_body(residual, Sp, u, dinv, hprev, bg, lng, lnb, wgn,
                 h_out, un_out):
    S = Sp[0, :_N, :] + Sp[1, :_N, :]
    agg = dinv[...] * (S + u[...]) + bg[...]
    h = _ln_relu(agg, lng[...], lnb[...])
    if residual:
        h = h + hprev[...]
    h_out[...] = h
    un_out[...] = dinv[...] * jnp.dot(h, wgn[...],
                                      preferred_element_type=jnp.float32)


def _tc_post_body(Sp, u, dinv, hprev, bg, lng, lnb, batch,
                  wh1, bh1, wh2, bh2, wh3, bh3, out):
    S = Sp[0, :_N, :] + Sp[1, :_N, :]
    agg = dinv[...] * (S + u[...]) + bg[...]
    h = _ln_relu(agg, lng[...], lnb[...]) + hprev[...]
    mask = (batch[...] == lax.broadcasted_iota(jnp.int32, (_G, _N), 0))
    mask = mask.astype(jnp.float32)
    counts = jnp.sum(mask, axis=1, keepdims=True)
    pooled = jnp.dot(mask, h, preferred_element_type=jnp.float32)
    pooled = pooled / jnp.maximum(counts, 1.0)
    x = jnp.maximum(jnp.dot(pooled, wh1[...],
                            preferred_element_type=jnp.float32) + bh1[...], 0.0)
    x = jnp.maximum(jnp.dot(x, wh2[...],
                            preferred_element_type=jnp.float32) + bh2[...], 0.0)
    y = jnp.dot(x, wh3[...], preferred_element_type=jnp.float32) + bh3[...]
    out[...] = jnp.maximum(y, 0.0) + jnp.log(1.0 + jnp.exp(-jnp.abs(y)))


_f32 = jnp.float32
_nh = jax.ShapeDtypeStruct((_N, _H), _f32)

_tc_pre = pl.pallas_call(_tc_pre_body,
                         out_shape=(_nh, jax.ShapeDtypeStruct((_N, 1), _f32)))
_tc_mid0 = pl.pallas_call(functools.partial(_tc_mid_body, False),
                          out_shape=(_nh, _nh))
_tc_mid1 = pl.pallas_call(functools.partial(_tc_mid_body, True),
                          out_shape=(_nh, _nh))
_tc_post = pl.pallas_call(_tc_post_body,
                          out_shape=jax.ShapeDtypeStruct((_G, 1), _f32))


def kernel(lightcurve, edge_index, batch,
           W_enc1, b_enc1, W_enc2, b_enc2,
           W_g0, b_g0, ln_g0, ln_b0,
           W_g1, b_g1, ln_g1, ln_b1,
           W_g2, b_g2, ln_g2, ln_b2,
           W_h1, b_h1, W_h2, b_h2, W_h3, b_h3):
    pad = _EPWP - _EPW
    src = edge_index[0].reshape(_NW, _EPW)
    dst = edge_index[1].reshape(_NW, _EPW)
    srcp = jnp.pad(src, ((0, 0), (0, pad))).reshape(_NW, _CPW, _CH)
    dstp = jnp.pad(dst, ((0, 0), (0, pad)),
                   constant_values=_N).reshape(_NW, _CPW, _CH)

    r = lambda a: a.reshape(1, -1)

    sc_deg, sc_msg = _sc_deg(), _sc_msg()
    degp = sc_deg(dstp)
    u0, dinv = _tc_pre(lightcurve, r(W_enc1), r(b_enc1), W_enc2, r(b_enc2),
                       W_g0, degp)
    S0 = sc_msg(u0, srcp, dstp)
    h1, u1 = _tc_mid0(S0, u0, dinv, u0, r(b_g0), r(ln_g0), r(ln_b0), W_g1)
    S1 = sc_msg(u1, srcp, dstp)
    h2, u2 = _tc_mid1(S1, u1, dinv, h1, r(b_g1), r(ln_g1), r(ln_b1), W_g2)
    S2 = sc_msg(u2, srcp, dstp)
    out = _tc_post(S2, u2, dinv, h2, r(b_g2), r(ln_g2), r(ln_b2), r(batch),
                   W_h1, r(b_h1), W_h2, r(b_h2), W_h3, r(b_h3))
    return out


# speedup vs baseline: 1.0013x; 1.0013x over previous
import functools

import jax
import jax.numpy as jnp
from jax import lax
from jax.experimental import pallas as pl
from jax.experimental.pallas import tpu as pltpu
from jax.experimental.pallas import tpu_sc as plsc

_N = 10000
_E = 320000
_H = 128
_G = 16

_NC = 2
_NS = 16
_NW = _NC * _NS
_CH = 128
_EPW = _E // _NW
_CPW = 80
_WCH = 40
_EPWP = _CPW * _CH
_NPAD = 10112
_RPT = _NPAD // _NS
_DW = 16

@functools.cache
def _mesh():
    return plsc.VectorSubcoreMesh(core_axis_name="c", subcore_axis_name="s",
                                  num_cores=_NC, num_subcores=_NS)


def _sc_deg_body(dstp_hbm, out_hbm, dst_v, ones_v, zb_v, acc_sh):
    c = lax.axis_index("c")
    s = lax.axis_index("s")
    wid = s * _NC + c

    lane = lax.iota(jnp.int32, 16)
    onehot = jnp.where(lane == 0, 1.0, 0.0).astype(jnp.float32)
    z16 = jnp.zeros((16,), jnp.float32)

    def fill(r, carry):
        ones_v[r, :] = onehot
        zb_v[r, :] = z16
        return carry

    lax.fori_loop(0, _CH, fill, 0)

    base = s * _RPT
    nfull = _RPT // _CH
    for k in range(nfull):
        pltpu.sync_copy(zb_v, acc_sh.at[pl.ds(base + k * _CH, _CH)])
    rem = _RPT - nfull * _CH
    if rem:
        pltpu.sync_copy(zb_v.at[pl.ds(0, rem)],
                        acc_sh.at[pl.ds(base + nfull * _CH, rem)])

    pltpu.sync_copy(dstp_hbm.at[wid], dst_v)
    plsc.subcore_barrier()

    def step(j, carry):
        pltpu.sync_copy(ones_v, acc_sh.at[dst_v.at[j]], add=True)
        return carry

    lax.fori_loop(0, _CPW, step, 0)
    plsc.subcore_barrier()
    pltpu.sync_copy(acc_sh.at[pl.ds(base, _RPT)],
                    out_hbm.at[c, pl.ds(base, _RPT)])


@functools.cache
def _sc_deg():
    return pl.kernel(
        _sc_deg_body,
        out_type=jax.ShapeDtypeStruct((_NC, _NPAD, _DW), jnp.float32),
        mesh=_mesh(),
        scratch_types=[
            pltpu.VMEM((_CPW, _CH), jnp.int32),
            pltpu.VMEM((_CH, _DW), jnp.float32),
            pltpu.VMEM((_CH, _DW), jnp.float32),
            pltpu.VMEM_SHARED((_NPAD, _DW), jnp.float32),
        ],
    )


def _sc_msg_body(u_hbm, srcp_hbm, dstp_hbm, out_hbm,
                 src_v, dst_v, rows0_v, acc_sh, sem_g0):
    c = lax.axis_index("c")
    s = lax.axis_index("s")
    wid = s * _NC + c

    z16 = jnp.zeros((16,), jnp.float32)

    def zfill(r, carry):
        for cc in range(_H // 16):
            rows0_v[r, pl.ds(cc * 16, 16)] = z16
        return carry

    lax.fori_loop(0, _CH, zfill, 0)

    base = s * _RPT
    nfull = _RPT // _CH
    for k in range(nfull):
        pltpu.sync_copy(rows0_v, acc_sh.at[pl.ds(base + k * _CH, _CH)])
    rem = _RPT - nfull * _CH
    if rem:
        pltpu.sync_copy(rows0_v.at[pl.ds(0, rem)],
                        acc_sh.at[pl.ds(base + nfull * _CH, rem)])

    plsc.subcore_barrier()

    pltpu.sync_copy(srcp_hbm.at[wid], src_v)
    pltpu.sync_copy(dstp_hbm.at[wid], dst_v)

    def step(j, carry):
        pltpu.async_copy(u_hbm.at[src_v.at[j]], rows0_v, sem_g0).wait()
        pltpu.sync_copy(rows0_v, acc_sh.at[dst_v.at[j]], add=True)
        return carry

    lax.fori_loop(0, _CPW, step, 0)

    plsc.subcore_barrier()
    pltpu.sync_copy(acc_sh.at[pl.ds(base, _RPT)],
                    out_hbm.at[c, pl.ds(base, _RPT)])


@functools.cache
def _sc_msg():
    return pl.kernel(
        _sc_msg_body,
        out_type=jax.ShapeDtypeStruct((_NC, _NPAD, _H), jnp.float32),
        mesh=_mesh(),
        scratch_types=[
            pltpu.VMEM((_CPW, _CH), jnp.int32),
            pltpu.VMEM((_CPW, _CH), jnp.int32),
            pltpu.VMEM((_CH, _H), jnp.float32),
            pltpu.VMEM_SHARED((_NPAD, _H), jnp.float32),
            pltpu.SemaphoreType.DMA,
        ],
    )


def _tc_pre_body(lc, w1, b1, w2, b2, wg0, degp, u0_out, dinv_out):
    deg = (jnp.sum(degp[0, :_N, :], axis=-1, keepdims=True)
           + jnp.sum(degp[1, :_N, :], axis=-1, keepdims=True) + 1.0)
    dinv = lax.rsqrt(deg)
    h = jnp.maximum(lc[...] * w1[...] + b1[...], 0.0)
    h = jnp.dot(h, w2[...], preferred_element_type=jnp.float32) + b2[...]
    u0_out[...] = dinv * jnp.dot(h, wg0[...], preferred_element_type=jnp.float32)
    dinv_out[...] = dinv


def _ln_relu(agg, lng, lnb):
    m = jnp.mean(agg, axis=-1, keepdims=True)
    v = jnp.mean((agg - m) ** 2, axis=-1, keepdims=True)
    h = (agg - m) * lax.rsqrt(v + 1e-5) * lng + lnb
    return jnp.maximum(h, 0.0)


def _tc_mid_body(residual, Sp, u, dinv, hprev, bg, lng, lnb, wgn,
                 h_out, un_out):
    S = Sp[0, :_N, :] + Sp[1, :_N, :]
    agg = dinv[...] * (S + u[...]) + bg[...]
    h = _ln_relu(agg, lng[...], lnb[...])
    if residual:
        h = h + hprev[...]
    h_out[...] = h
    un_out[...] = dinv[...] * jnp.dot(h, wgn[...],
                                      preferred_element_type=jnp.float32)


def _tc_post_body(Sp, u, dinv, hprev, bg, lng, lnb, batch,
                  wh1, bh1, wh2, bh2, wh3, bh3, out):
    S = Sp[0, :_N, :] + Sp[1, :_N, :]
    agg = dinv[...] * (S + u[...]) + bg[...]
    h = _ln_relu(agg, lng[...], lnb[...]) + hprev[...]
    mask = (batch[...] == lax.broadcasted_iota(jnp.int32, (_G, _N), 0))
    mask = mask.astype(jnp.float32)
    counts = jnp.sum(mask, axis=1, keepdims=True)
    pooled = jnp.dot(mask, h, preferred_element_type=jnp.float32)
    pooled = pooled / jnp.maximum(counts, 1.0)
    x = jnp.maximum(jnp.dot(pooled, wh1[...],
                            preferred_element_type=jnp.float32) + bh1[...], 0.0)
    x = jnp.maximum(jnp.dot(x, wh2[...],
                            preferred_element_type=jnp.float32) + bh2[...], 0.0)
    y = jnp.dot(x, wh3[...], preferred_element_type=jnp.float32) + bh3[...]
    out[...] = jnp.maximum(y, 0.0) + jnp.log(1.0 + jnp.exp(-jnp.abs(y)))


_f32 = jnp.float32
_nh = jax.ShapeDtypeStruct((_N, _H), _f32)

_tc_pre = pl.pallas_call(_tc_pre_body,
                         out_shape=(_nh, jax.ShapeDtypeStruct((_N, 1), _f32)))
_tc_mid0 = pl.pallas_call(functools.partial(_tc_mid_body, False),
                          out_shape=(_nh, _nh))
_tc_mid1 = pl.pallas_call(functools.partial(_tc_mid_body, True),
                          out_shape=(_nh, _nh))
_tc_post = pl.pallas_call(_tc_post_body,
                          out_shape=jax.ShapeDtypeStruct((_G, 1), _f32))


def kernel(lightcurve, edge_index, batch,
           W_enc1, b_enc1, W_enc2, b_enc2,
           W_g0, b_g0, ln_g0, ln_b0,
           W_g1, b_g1, ln_g1, ln_b1,
           W_g2, b_g2, ln_g2, ln_b2,
           W_h1, b_h1, W_h2, b_h2, W_h3, b_h3):
    pad = _EPWP - _EPW
    src = edge_index[0].reshape(_NW, _EPW)
    dst = edge_index[1].reshape(_NW, _EPW)
    srcp = jnp.pad(src, ((0, 0), (0, pad))).reshape(_NW, _CPW, _CH)
    sink = _N + (jnp.arange(pad, dtype=jnp.int32) % (_NPAD - _N))
    dstp = jnp.concatenate(
        [dst, jnp.broadcast_to(sink, (_NW, pad))],
        axis=1).reshape(_NW, _CPW, _CH)

    r = lambda a: a.reshape(1, -1)

    sc_deg, sc_msg = _sc_deg(), _sc_msg()
    degp = sc_deg(dstp)
    u0, dinv = _tc_pre(lightcurve, r(W_enc1), r(b_enc1), W_enc2, r(b_enc2),
                       W_g0, degp)
    S0 = sc_msg(u0, srcp, dstp)
    h1, u1 = _tc_mid0(S0, u0, dinv, u0, r(b_g0), r(ln_g0), r(ln_b0), W_g1)
    S1 = sc_msg(u1, srcp, dstp)
    h2, u2 = _tc_mid1(S1, u1, dinv, h1, r(b_g1), r(ln_g1), r(ln_b1), W_g2)
    S2 = sc_msg(u2, srcp, dstp)
    out = _tc_post(S2, u2, dinv, h2, r(b_g2), r(ln_g2), r(ln_b2), r(batch),
                   W_h1, r(b_h1), W_h2, r(b_h2), W_h3, r(b_h3))
    return out


# speedup vs baseline: 2.3112x; 2.3082x over previous
import functools

import jax
import jax.numpy as jnp
from jax import lax
from jax.experimental import pallas as pl
from jax.experimental.pallas import tpu as pltpu
from jax.experimental.pallas import tpu_sc as plsc

_N = 10000
_E = 320000
_H = 128
_G = 16

_NC = 2
_NS = 16
_NW = _NC * _NS
_CH = 128
_EPW = _E // _NW
_CPW = 80
_WCH = 40
_EPWP = _CPW * _CH
_NPAD = 10112
_RPT = _NPAD // _NS
_DW = 16

@functools.cache
def _mesh():
    return plsc.VectorSubcoreMesh(core_axis_name="c", subcore_axis_name="s",
                                  num_cores=_NC, num_subcores=_NS)


def _sc_deg_body(dstp_hbm, out_hbm, dst_v, ones_v, zb_v, acc_sh):
    c = lax.axis_index("c")
    s = lax.axis_index("s")
    wid = s * _NC + c

    lane = lax.iota(jnp.int32, 16)
    onehot = jnp.where(lane == 0, 1.0, 0.0).astype(jnp.float32)
    z16 = jnp.zeros((16,), jnp.float32)

    def fill(r, carry):
        ones_v[r, :] = onehot
        zb_v[r, :] = z16
        return carry

    lax.fori_loop(0, _CH, fill, 0)

    base = s * _RPT
    nfull = _RPT // _CH
    for k in range(nfull):
        pltpu.sync_copy(zb_v, acc_sh.at[pl.ds(base + k * _CH, _CH)])
    rem = _RPT - nfull * _CH
    if rem:
        pltpu.sync_copy(zb_v.at[pl.ds(0, rem)],
                        acc_sh.at[pl.ds(base + nfull * _CH, rem)])

    pltpu.sync_copy(dstp_hbm.at[wid], dst_v)
    plsc.subcore_barrier()

    def step(j, carry):
        pltpu.sync_copy(ones_v, acc_sh.at[dst_v.at[j]], add=True)
        return carry

    lax.fori_loop(0, _CPW, step, 0)
    plsc.subcore_barrier()
    pltpu.sync_copy(acc_sh.at[pl.ds(base, _RPT)],
                    out_hbm.at[c, pl.ds(base, _RPT)])


@functools.cache
def _sc_deg():
    return pl.kernel(
        _sc_deg_body,
        out_type=jax.ShapeDtypeStruct((_NC, _NPAD, _DW), jnp.float32),
        mesh=_mesh(),
        scratch_types=[
            pltpu.VMEM((_CPW, _CH), jnp.int32),
            pltpu.VMEM((_CH, _DW), jnp.float32),
            pltpu.VMEM((_CH, _DW), jnp.float32),
            pltpu.VMEM_SHARED((_NPAD, _DW), jnp.float32),
        ],
    )


def _sc_msg_body(u_hbm, srcp_hbm, dstp_hbm, out_hbm,
                 src_v, dst_v, rows0_v, acc_sh, sem_g0):
    c = lax.axis_index("c")
    s = lax.axis_index("s")
    wid = s * _NC + c

    z16 = jnp.zeros((16,), jnp.float32)

    def zfill(r, carry):
        for cc in range(_H // 16):
            rows0_v[r, pl.ds(cc * 16, 16)] = z16
        return carry

    lax.fori_loop(0, _CH, zfill, 0)

    base = s * _RPT
    nfull = _RPT // _CH
    for k in range(nfull):
        pltpu.sync_copy(rows0_v, acc_sh.at[pl.ds(base + k * _CH, _CH)])
    rem = _RPT - nfull * _CH
    if rem:
        pltpu.sync_copy(rows0_v.at[pl.ds(0, rem)],
                        acc_sh.at[pl.ds(base + nfull * _CH, rem)])

    plsc.subcore_barrier()

    pltpu.sync_copy(srcp_hbm.at[wid], src_v)
    pltpu.sync_copy(dstp_hbm.at[wid], dst_v)

    def step(j, carry):
        pltpu.async_copy(u_hbm.at[src_v.at[j]], rows0_v, sem_g0).wait()
        pltpu.sync_copy(rows0_v, acc_sh.at[dst_v.at[j]], add=True)
        return carry

    lax.fori_loop(0, _CPW, step, 0)

    plsc.subcore_barrier()
    pltpu.sync_copy(acc_sh.at[pl.ds(base, _RPT)],
                    out_hbm.at[c, pl.ds(base, _RPT)])


@functools.cache
def _sc_msg():
    return pl.kernel(
        _sc_msg_body,
        out_type=jax.ShapeDtypeStruct((_NC, _NPAD, _H), jnp.float32),
        mesh=_mesh(),
        scratch_types=[
            pltpu.VMEM((_CPW, _CH), jnp.int32),
            pltpu.VMEM((_CPW, _CH), jnp.int32),
            pltpu.VMEM((_CH, _H), jnp.float32),
            pltpu.VMEM_SHARED((_NPAD, _H), jnp.float32),
            pltpu.SemaphoreType.DMA,
        ],
    )


def _tc_pre_body(lc, w1, b1, w2, b2, wg0, degp, u0_out, dinv_out):
    deg = (jnp.sum(degp[0, :_N, :], axis=-1, keepdims=True)
           + jnp.sum(degp[1, :_N, :], axis=-1, keepdims=True) + 1.0)
    dinv = lax.rsqrt(deg)
    h = jnp.maximum(lc[...] * w1[...] + b1[...], 0.0)
    h = jnp.dot(h, w2[...], preferred_element_type=jnp.float32) + b2[...]
    u0_out[...] = dinv * jnp.dot(h, wg0[...], preferred_element_type=jnp.float32)
    dinv_out[...] = dinv


def _ln_relu(agg, lng, lnb):
    m = jnp.mean(agg, axis=-1, keepdims=True)
    v = jnp.mean((agg - m) ** 2, axis=-1, keepdims=True)
    h = (agg - m) * lax.rsqrt(v + 1e-5) * lng + lnb
    return jnp.maximum(h, 0.0)


def _tc_mid_body(residual, Sp, u, dinv, hprev, bg, lng, lnb, wgn,
                 h_out, un_out):
    S = Sp[0, :_N, :] + Sp[1, :_N, :]
    agg = dinv[...] * (S + u[...]) + bg[...]
    h = _ln_relu(agg, lng[...], lnb[...])
    if residual:
        h = h + hprev[...]
    h_out[...] = h
    un_out[...] = dinv[...] * jnp.dot(h, wgn[...],
                                      preferred_element_type=jnp.float32)


def _tc_post_body(Sp, u, dinv, hprev, bg, lng, lnb, batch,
                  wh1, bh1, wh2, bh2, wh3, bh3, out):
    S = Sp[0, :_N, :] + Sp[1, :_N, :]
    agg = dinv[...] * (S + u[...]) + bg[...]
    h = _ln_relu(agg, lng[...], lnb[...]) + hprev[...]
    mask = (batch[...] == lax.broadcasted_iota(jnp.int32, (_G, _N), 0))
    mask = mask.astype(jnp.float32)
    counts = jnp.sum(mask, axis=1, keepdims=True)
    pooled = jnp.dot(mask, h, preferred_element_type=jnp.float32)
    pooled = pooled / jnp.maximum(counts, 1.0)
    x = jnp.maximum(jnp.dot(pooled, wh1[...],
                            preferred_element_type=jnp.float32) + bh1[...], 0.0)
    x = jnp.maximum(jnp.dot(x, wh2[...],
                            preferred_element_type=jnp.float32) + bh2[...], 0.0)
    y = jnp.dot(x, wh3[...], preferred_element_type=jnp.float32) + bh3[...]
    out[...] = jnp.maximum(y, 0.0) + jnp.log(1.0 + jnp.exp(-jnp.abs(y)))


_f32 = jnp.float32
_nh = jax.ShapeDtypeStruct((_N, _H), _f32)

_tc_pre = pl.pallas_call(_tc_pre_body,
                         out_shape=(_nh, jax.ShapeDtypeStruct((_N, 1), _f32)))
_tc_mid0 = pl.pallas_call(functools.partial(_tc_mid_body, False),
                          out_shape=(_nh, _nh))
_tc_mid1 = pl.pallas_call(functools.partial(_tc_mid_body, True),
                          out_shape=(_nh, _nh))
_tc_post = pl.pallas_call(_tc_post_body,
                          out_shape=jax.ShapeDtypeStruct((_G, 1), _f32))


def kernel(lightcurve, edge_index, batch,
           W_enc1, b_enc1, W_enc2, b_enc2,
           W_g0, b_g0, ln_g0, ln_b0,
           W_g1, b_g1, ln_g1, ln_b1,
           W_g2, b_g2, ln_g2, ln_b2,
           W_h1, b_h1, W_h2, b_h2, W_h3, b_h3):
    pad = _EPWP - _EPW
    src = edge_index[0].reshape(_NW, _EPW)
    dst = edge_index[1].reshape(_NW, _EPW)
    fake = jnp.arange(pad, dtype=jnp.int32) * 41 % _N
    srcp = jnp.concatenate(
        [src, jnp.broadcast_to(fake, (_NW, pad))],
        axis=1).reshape(_NW, _CPW, _CH)
    sink = _N + (jnp.arange(pad, dtype=jnp.int32) % (_NPAD - _N))
    dstp = jnp.concatenate(
        [dst, jnp.broadcast_to(sink, (_NW, pad))],
        axis=1).reshape(_NW, _CPW, _CH)

    r = lambda a: a.reshape(1, -1)

    sc_deg, sc_msg = _sc_deg(), _sc_msg()
    degp = sc_deg(dstp)
    u0, dinv = _tc_pre(lightcurve, r(W_enc1), r(b_enc1), W_enc2, r(b_enc2),
                       W_g0, degp)
    S0 = sc_msg(u0, srcp, dstp)
    h1, u1 = _tc_mid0(S0, u0, dinv, u0, r(b_g0), r(ln_g0), r(ln_b0), W_g1)
    S1 = sc_msg(u1, srcp, dstp)
    h2, u2 = _tc_mid1(S1, u1, dinv, h1, r(b_g1), r(ln_g1), r(ln_b1), W_g2)
    S2 = sc_msg(u2, srcp, dstp)
    out = _tc_post(S2, u2, dinv, h2, r(b_g2), r(ln_g2), r(ln_b2), r(batch),
                   W_h1, r(b_h1), W_h2, r(b_h2), W_h3, r(b_h3))
    return out


# speedup vs baseline: 3.3379x; 1.4442x over previous
import functools

import jax
import jax.numpy as jnp
from jax import lax
from jax.experimental import pallas as pl
from jax.experimental.pallas import tpu as pltpu
from jax.experimental.pallas import tpu_sc as plsc

_N = 10000
_E = 320000
_H = 128
_G = 16

_NC = 2
_NS = 16
_NW = _NC * _NS
_CH = 128
_EPW = _E // _NW
_CPW = 80
_WCH = 40
_EPWP = _CPW * _CH
_NPAD = 10112
_RPT = _NPAD // _NS
_DW = 16

@functools.cache
def _mesh():
    return plsc.VectorSubcoreMesh(core_axis_name="c", subcore_axis_name="s",
                                  num_cores=_NC, num_subcores=_NS)


def _sc_deg_body(dstp_hbm, out_hbm, dst_v, ones_v, zb_v, acc_sh):
    c = lax.axis_index("c")
    s = lax.axis_index("s")
    wid = s * _NC + c

    lane = lax.iota(jnp.int32, 16)
    onehot = jnp.where(lane == 0, 1.0, 0.0).astype(jnp.float32)
    z16 = jnp.zeros((16,), jnp.float32)

    def fill(r, carry):
        ones_v[r, :] = onehot
        zb_v[r, :] = z16
        return carry

    lax.fori_loop(0, _CH, fill, 0)

    base = s * _RPT
    nfull = _RPT // _CH
    for k in range(nfull):
        pltpu.sync_copy(zb_v, acc_sh.at[pl.ds(base + k * _CH, _CH)])
    rem = _RPT - nfull * _CH
    if rem:
        pltpu.sync_copy(zb_v.at[pl.ds(0, rem)],
                        acc_sh.at[pl.ds(base + nfull * _CH, rem)])

    pltpu.sync_copy(dstp_hbm.at[wid], dst_v)
    plsc.subcore_barrier()

    def step(j, carry):
        pltpu.sync_copy(ones_v, acc_sh.at[dst_v.at[j]], add=True)
        return carry

    lax.fori_loop(0, _CPW, step, 0)
    plsc.subcore_barrier()
    pltpu.sync_copy(acc_sh.at[pl.ds(base, _RPT)],
                    out_hbm.at[c, pl.ds(base, _RPT)])


@functools.cache
def _sc_deg():
    return pl.kernel(
        _sc_deg_body,
        out_type=jax.ShapeDtypeStruct((_NC, _NPAD, _DW), jnp.float32),
        mesh=_mesh(),
        scratch_types=[
            pltpu.VMEM((_CPW, _CH), jnp.int32),
            pltpu.VMEM((_CH, _DW), jnp.float32),
            pltpu.VMEM((_CH, _DW), jnp.float32),
            pltpu.VMEM_SHARED((_NPAD, _DW), jnp.float32),
        ],
    )


def _sc_msg_body(u_hbm, srcp_hbm, dstp_hbm, out_hbm,
                 src_v, dst_v, rows0_v, acc_sh, sem_g0, sem_g1):
    c = lax.axis_index("c")
    s = lax.axis_index("s")
    wid = s * _NC + c

    z16 = jnp.zeros((16,), jnp.float32)

    def zfill(r, carry):
        for cc in range(_H // 16):
            rows0_v[0, r, pl.ds(cc * 16, 16)] = z16
        return carry

    lax.fori_loop(0, _CH, zfill, 0)

    base = s * _RPT
    nfull = _RPT // _CH
    for k in range(nfull):
        pltpu.sync_copy(rows0_v.at[0], acc_sh.at[pl.ds(base + k * _CH, _CH)])
    rem = _RPT - nfull * _CH
    if rem:
        pltpu.sync_copy(rows0_v.at[0, pl.ds(0, rem)],
                        acc_sh.at[pl.ds(base + nfull * _CH, rem)])

    plsc.subcore_barrier()

    def gather(j, p, sem):
        return pltpu.async_copy(u_hbm.at[src_v.at[j]], rows0_v.at[p], sem)

    def wait_gather(p, sem):
        pltpu.make_async_copy(u_hbm.at[src_v.at[0]], rows0_v.at[p], sem).wait()

    def scatter(j, p):
        pltpu.sync_copy(rows0_v.at[p], acc_sh.at[dst_v.at[j]], add=True)

    def pair(k, carry):
        j0 = 2 * k
        gather(j0 + 1, 1, sem_g1)
        wait_gather(0, sem_g0)
        scatter(j0, 0)
        gather(j0 + 2, 0, sem_g0)
        wait_gather(1, sem_g1)
        scatter(j0 + 1, 1)
        return carry

    for h in range(2):
        pltpu.sync_copy(srcp_hbm.at[wid, pl.ds(h * _WCH, _WCH)], src_v)
        pltpu.sync_copy(dstp_hbm.at[wid, pl.ds(h * _WCH, _WCH)], dst_v)
        gather(0, 0, sem_g0)
        lax.fori_loop(0, _WCH // 2 - 1, pair, 0)
        gather(_WCH - 1, 1, sem_g1)
        wait_gather(0, sem_g0)
        scatter(_WCH - 2, 0)
        wait_gather(1, sem_g1)
        scatter(_WCH - 1, 1)

    plsc.subcore_barrier()
    pltpu.sync_copy(acc_sh.at[pl.ds(base, _RPT)],
                    out_hbm.at[c, pl.ds(base, _RPT)])


@functools.cache
def _sc_msg():
    return pl.kernel(
        _sc_msg_body,
        out_type=jax.ShapeDtypeStruct((_NC, _NPAD, _H), jnp.float32),
        mesh=_mesh(),
        scratch_types=[
            pltpu.VMEM((_WCH, _CH), jnp.int32),
            pltpu.VMEM((_WCH, _CH), jnp.int32),
            pltpu.VMEM((2, _CH, _H), jnp.float32),
            pltpu.VMEM_SHARED((_NPAD, _H), jnp.float32),
            pltpu.SemaphoreType.DMA,
            pltpu.SemaphoreType.DMA,
        ],
    )


def _tc_pre_body(lc, w1, b1, w2, b2, wg0, degp, u0_out, dinv_out):
    deg = (jnp.sum(degp[0, :_N, :], axis=-1, keepdims=True)
           + jnp.sum(degp[1, :_N, :], axis=-1, keepdims=True) + 1.0)
    dinv = lax.rsqrt(deg)
    h = jnp.maximum(lc[...] * w1[...] + b1[...], 0.0)
    h = jnp.dot(h, w2[...], preferred_element_type=jnp.float32) + b2[...]
    u0_out[...] = dinv * jnp.dot(h, wg0[...], preferred_element_type=jnp.float32)
    dinv_out[...] = dinv


def _ln_relu(agg, lng, lnb):
    m = jnp.mean(agg, axis=-1, keepdims=True)
    v = jnp.mean((agg - m) ** 2, axis=-1, keepdims=True)
    h = (agg - m) * lax.rsqrt(v + 1e-5) * lng + lnb
    return jnp.maximum(h, 0.0)


def _tc_mid_body(residual, Sp, u, dinv, hprev, bg, lng, lnb, wgn,
                 h_out, un_out):
    S = Sp[0, :_N, :] + Sp[1, :_N, :]
    agg = dinv[...] * (S + u[...]) + bg[...]
    h = _ln_relu(agg, lng[...], lnb[...])
    if residual:
        h = h + hprev[...]
    h_out[...] = h
    un_out[...] = dinv[...] * jnp.dot(h, wgn[...],
                                      preferred_element_type=jnp.float32)


def _tc_post_body(Sp, u, dinv, hprev, bg, lng, lnb, batch,
                  wh1, bh1, wh2, bh2, wh3, bh3, out):
    S = Sp[0, :_N, :] + Sp[1, :_N, :]
    agg = dinv[...] * (S + u[...]) + bg[...]
    h = _ln_relu(agg, lng[...], lnb[...]) + hprev[...]
    mask = (batch[...] == lax.broadcasted_iota(jnp.int32, (_G, _N), 0))
    mask = mask.astype(jnp.float32)
    counts = jnp.sum(mask, axis=1, keepdims=True)
    pooled = jnp.dot(mask, h, preferred_element_type=jnp.float32)
    pooled = pooled / jnp.maximum(counts, 1.0)
    x = jnp.maximum(jnp.dot(pooled, wh1[...],
                            preferred_element_type=jnp.float32) + bh1[...], 0.0)
    x = jnp.maximum(jnp.dot(x, wh2[...],
                            preferred_element_type=jnp.float32) + bh2[...], 0.0)
    y = jnp.dot(x, wh3[...], preferred_element_type=jnp.float32) + bh3[...]
    out[...] = jnp.maximum(y, 0.0) + jnp.log(1.0 + jnp.exp(-jnp.abs(y)))


_f32 = jnp.float32
_nh = jax.ShapeDtypeStruct((_N, _H), _f32)

_tc_pre = pl.pallas_call(_tc_pre_body,
                         out_shape=(_nh, jax.ShapeDtypeStruct((_N, 1), _f32)))
_tc_mid0 = pl.pallas_call(functools.partial(_tc_mid_body, False),
                          out_shape=(_nh, _nh))
_tc_mid1 = pl.pallas_call(functools.partial(_tc_mid_body, True),
                          out_shape=(_nh, _nh))
_tc_post = pl.pallas_call(_tc_post_body,
                          out_shape=jax.ShapeDtypeStruct((_G, 1), _f32))


def kernel(lightcurve, edge_index, batch,
           W_enc1, b_enc1, W_enc2, b_enc2,
           W_g0, b_g0, ln_g0, ln_b0,
           W_g1, b_g1, ln_g1, ln_b1,
           W_g2, b_g2, ln_g2, ln_b2,
           W_h1, b_h1, W_h2, b_h2, W_h3, b_h3):
    pad = _EPWP - _EPW
    src = edge_index[0].reshape(_NW, _EPW)
    dst = edge_index[1].reshape(_NW, _EPW)
    fake = jnp.arange(pad, dtype=jnp.int32) * 41 % _N
    srcp = jnp.concatenate(
        [src, jnp.broadcast_to(fake, (_NW, pad))],
        axis=1).reshape(_NW, _CPW, _CH)
    sink = _N + (jnp.arange(pad, dtype=jnp.int32) % (_NPAD - _N))
    dstp = jnp.concatenate(
        [dst, jnp.broadcast_to(sink, (_NW, pad))],
        axis=1).reshape(_NW, _CPW, _CH)

    r = lambda a: a.reshape(1, -1)

    sc_deg, sc_msg = _sc_deg(), _sc_msg()
    degp = sc_deg(dstp)
    u0, dinv = _tc_pre(lightcurve, r(W_enc1), r(b_enc1), W_enc2, r(b_enc2),
                       W_g0, degp)
    S0 = sc_msg(u0, srcp, dstp)
    h1, u1 = _tc_mid0(S0, u0, dinv, u0, r(b_g0), r(ln_g0), r(ln_b0), W_g1)
    S1 = sc_msg(u1, srcp, dstp)
    h2, u2 = _tc_mid1(S1, u1, dinv, h1, r(b_g1), r(ln_g1), r(ln_b1), W_g2)
    S2 = sc_msg(u2, srcp, dstp)
    out = _tc_post(S2, u2, dinv, h2, r(b_g2), r(ln_g2), r(ln_b2), r(batch),
                   W_h1, r(b_h1), W_h2, r(b_h2), W_h3, r(b_h3))
    return out


# speedup vs baseline: 3.3416x; 1.0011x over previous
import functools

import jax
import jax.numpy as jnp
from jax import lax
from jax.experimental import pallas as pl
from jax.experimental.pallas import tpu as pltpu
from jax.experimental.pallas import tpu_sc as plsc

_N = 10000
_E = 320000
_H = 128
_G = 16

_NC = 2
_NS = 16
_NW = _NC * _NS
_CH = 128
_EPW = _E // _NW
_CPW = 80
_WCH = 40
_EPWP = _CPW * _CH
_NPAD = 10112
_RPT = _NPAD // _NS
_DW = 16

@functools.cache
def _mesh():
    return plsc.VectorSubcoreMesh(core_axis_name="c", subcore_axis_name="s",
                                  num_cores=_NC, num_subcores=_NS)


def _sc_deg_body(dstp_hbm, out_hbm, dst_v, ones_v, zb_v, acc_sh):
    c = lax.axis_index("c")
    s = lax.axis_index("s")
    wid = s * _NC + c

    lane = lax.iota(jnp.int32, 16)
    onehot = jnp.where(lane == 0, 1.0, 0.0).astype(jnp.float32)
    z16 = jnp.zeros((16,), jnp.float32)

    def fill(r, carry):
        ones_v[r, :] = onehot
        zb_v[r, :] = z16
        return carry

    lax.fori_loop(0, _CH, fill, 0)

    base = s * _RPT
    nfull = _RPT // _CH
    for k in range(nfull):
        pltpu.sync_copy(zb_v, acc_sh.at[pl.ds(base + k * _CH, _CH)])
    rem = _RPT - nfull * _CH
    if rem:
        pltpu.sync_copy(zb_v.at[pl.ds(0, rem)],
                        acc_sh.at[pl.ds(base + nfull * _CH, rem)])

    pltpu.sync_copy(dstp_hbm.at[wid], dst_v)
    plsc.subcore_barrier()

    def step(j, carry):
        pltpu.sync_copy(ones_v, acc_sh.at[dst_v.at[j]], add=True)
        return carry

    lax.fori_loop(0, _CPW, step, 0)
    plsc.subcore_barrier()
    pltpu.sync_copy(acc_sh.at[pl.ds(base, _RPT)],
                    out_hbm.at[c, pl.ds(base, _RPT)])


@functools.cache
def _sc_deg():
    return pl.kernel(
        _sc_deg_body,
        out_type=jax.ShapeDtypeStruct((_NC, _NPAD, _DW), jnp.float32),
        mesh=_mesh(),
        scratch_types=[
            pltpu.VMEM((_CPW, _CH), jnp.int32),
            pltpu.VMEM((_CH, _DW), jnp.float32),
            pltpu.VMEM((_CH, _DW), jnp.float32),
            pltpu.VMEM_SHARED((_NPAD, _DW), jnp.float32),
        ],
    )


def _sc_msg_body(u_hbm, srcp_hbm, dstp_hbm, out_hbm,
                 src_v, dst_v, rows0_v, acc_sh, sem_g0, sem_g1):
    c = lax.axis_index("c")
    s = lax.axis_index("s")
    wid = s * _NC + c

    z16 = jnp.zeros((16,), jnp.float32)

    def zfill(r, carry):
        for cc in range(_H // 16):
            rows0_v[0, r, pl.ds(cc * 16, 16)] = z16
        return carry

    lax.fori_loop(0, _CH, zfill, 0)

    base = s * _RPT
    nfull = _RPT // _CH
    for k in range(nfull):
        pltpu.sync_copy(rows0_v.at[0], acc_sh.at[pl.ds(base + k * _CH, _CH)])
    rem = _RPT - nfull * _CH
    if rem:
        pltpu.sync_copy(rows0_v.at[0, pl.ds(0, rem)],
                        acc_sh.at[pl.ds(base + nfull * _CH, rem)])

    plsc.subcore_barrier()

    def gather(j, p, sem):
        return pltpu.async_copy(u_hbm.at[src_v.at[j]], rows0_v.at[p], sem)

    def wait_gather(p, sem):
        pltpu.make_async_copy(u_hbm.at[src_v.at[0]], rows0_v.at[p], sem).wait()

    def scatter(j, p):
        pltpu.sync_copy(rows0_v.at[p], acc_sh.at[dst_v.at[j]], add=True)

    def pair(k, carry):
        j0 = 2 * k
        gather(j0 + 1, 1, sem_g1)
        wait_gather(0, sem_g0)
        scatter(j0, 0)
        gather(j0 + 2, 0, sem_g0)
        wait_gather(1, sem_g1)
        scatter(j0 + 1, 1)
        return carry

    for h in range(2):
        pltpu.sync_copy(srcp_hbm.at[wid, pl.ds(h * _WCH, _WCH)], src_v)
        pltpu.sync_copy(dstp_hbm.at[wid, pl.ds(h * _WCH, _WCH)], dst_v)
        gather(0, 0, sem_g0)
        lax.fori_loop(0, _WCH // 2 - 1, pair, 0)
        gather(_WCH - 1, 1, sem_g1)
        wait_gather(0, sem_g0)
        scatter(_WCH - 2, 0)
        wait_gather(1, sem_g1)
        scatter(_WCH - 1, 1)

    plsc.subcore_barrier()
    pltpu.sync_copy(acc_sh.at[pl.ds(base, _RPT)],
                    out_hbm.at[c, pl.ds(base, _RPT)])


@functools.cache
def _sc_msg():
    return pl.kernel(
        _sc_msg_body,
        out_type=jax.ShapeDtypeStruct((_NC, _NPAD, _H), jnp.float32),
        mesh=_mesh(),
        scratch_types=[
            pltpu.VMEM((_WCH, _CH), jnp.int32),
            pltpu.VMEM((_WCH, _CH), jnp.int32),
            pltpu.VMEM((2, _CH, _H), jnp.float32),
            pltpu.VMEM_SHARED((_NPAD, _H), jnp.float32),
            pltpu.SemaphoreType.DMA,
            pltpu.SemaphoreType.DMA,
        ],
    )


def _tc_enc_body(lc, w1, b1, w2, b2, h_out):
    h = jnp.maximum(lc[...] * w1[...] + b1[...], 0.0)
    h_out[...] = jnp.dot(h, w2[...], preferred_element_type=jnp.float32) + b2[...]


def _tc_pre_body(henc, wg0, degp, u0_out, dinv_out):
    deg = (jnp.sum(degp[0, :_N, :], axis=-1, keepdims=True)
           + jnp.sum(degp[1, :_N, :], axis=-1, keepdims=True) + 1.0)
    dinv = lax.rsqrt(deg)
    u0_out[...] = dinv * jnp.dot(henc[...], wg0[...],
                                 preferred_element_type=jnp.float32)
    dinv_out[...] = dinv


def _ln_relu(agg, lng, lnb):
    m = jnp.mean(agg, axis=-1, keepdims=True)
    v = jnp.mean((agg - m) ** 2, axis=-1, keepdims=True)
    h = (agg - m) * lax.rsqrt(v + 1e-5) * lng + lnb
    return jnp.maximum(h, 0.0)


def _tc_mid_body(residual, Sp, u, dinv, hprev, bg, lng, lnb, wgn,
                 h_out, un_out):
    S = Sp[0, :_N, :] + Sp[1, :_N, :]
    agg = dinv[...] * (S + u[...]) + bg[...]
    h = _ln_relu(agg, lng[...], lnb[...])
    if residual:
        h = h + hprev[...]
    h_out[...] = h
    un_out[...] = dinv[...] * jnp.dot(h, wgn[...],
                                      preferred_element_type=jnp.float32)


def _tc_post_body(Sp, u, dinv, hprev, bg, lng, lnb, batch,
                  wh1, bh1, wh2, bh2, wh3, bh3, out):
    S = Sp[0, :_N, :] + Sp[1, :_N, :]
    agg = dinv[...] * (S + u[...]) + bg[...]
    h = _ln_relu(agg, lng[...], lnb[...]) + hprev[...]
    mask = (batch[...] == lax.broadcasted_iota(jnp.int32, (_G, _N), 0))
    mask = mask.astype(jnp.float32)
    counts = jnp.sum(mask, axis=1, keepdims=True)
    pooled = jnp.dot(mask, h, preferred_element_type=jnp.float32)
    pooled = pooled / jnp.maximum(counts, 1.0)
    x = jnp.maximum(jnp.dot(pooled, wh1[...],
                            preferred_element_type=jnp.float32) + bh1[...], 0.0)
    x = jnp.maximum(jnp.dot(x, wh2[...],
                            preferred_element_type=jnp.float32) + bh2[...], 0.0)
    y = jnp.dot(x, wh3[...], preferred_element_type=jnp.float32) + bh3[...]
    out[...] = jnp.maximum(y, 0.0) + jnp.log(1.0 + jnp.exp(-jnp.abs(y)))


_f32 = jnp.float32
_nh = jax.ShapeDtypeStruct((_N, _H), _f32)

_tc_enc = pl.pallas_call(_tc_enc_body, out_shape=_nh)
_tc_pre = pl.pallas_call(_tc_pre_body,
                         out_shape=(_nh, jax.ShapeDtypeStruct((_N, 1), _f32)))
_tc_mid0 = pl.pallas_call(functools.partial(_tc_mid_body, False),
                          out_shape=(_nh, _nh))
_tc_mid1 = pl.pallas_call(functools.partial(_tc_mid_body, True),
                          out_shape=(_nh, _nh))
_tc_post = pl.pallas_call(_tc_post_body,
                          out_shape=jax.ShapeDtypeStruct((_G, 1), _f32))


def kernel(lightcurve, edge_index, batch,
           W_enc1, b_enc1, W_enc2, b_enc2,
           W_g0, b_g0, ln_g0, ln_b0,
           W_g1, b_g1, ln_g1, ln_b1,
           W_g2, b_g2, ln_g2, ln_b2,
           W_h1, b_h1, W_h2, b_h2, W_h3, b_h3):
    pad = _EPWP - _EPW
    src = edge_index[0].reshape(_NW, _EPW)
    dst = edge_index[1].reshape(_NW, _EPW)
    fake = jnp.arange(pad, dtype=jnp.int32) * 41 % _N
    srcp = jnp.concatenate(
        [src, jnp.broadcast_to(fake, (_NW, pad))],
        axis=1).reshape(_NW, _CPW, _CH)
    sink = _N + (jnp.arange(pad, dtype=jnp.int32) % (_NPAD - _N))
    dstp = jnp.concatenate(
        [dst, jnp.broadcast_to(sink, (_NW, pad))],
        axis=1).reshape(_NW, _CPW, _CH)

    r = lambda a: a.reshape(1, -1)

    sc_deg, sc_msg = _sc_deg(), _sc_msg()
    degp = sc_deg(dstp)
    henc = _tc_enc(lightcurve, r(W_enc1), r(b_enc1), W_enc2, r(b_enc2))
    u0, dinv = _tc_pre(henc, W_g0, degp)
    S0 = sc_msg(u0, srcp, dstp)
    h1, u1 = _tc_mid0(S0, u0, dinv, u0, r(b_g0), r(ln_g0), r(ln_b0), W_g1)
    S1 = sc_msg(u1, srcp, dstp)
    h2, u2 = _tc_mid1(S1, u1, dinv, h1, r(b_g1), r(ln_g1), r(ln_b1), W_g2)
    S2 = sc_msg(u2, srcp, dstp)
    out = _tc_post(S2, u2, dinv, h2, r(b_g2), r(ln_g2), r(ln_b2), r(batch),
                   W_h1, r(b_h1), W_h2, r(b_h2), W_h3, r(b_h3))
    return out
